# Initial kernel scaffold; baseline (speedup 1.0000x reference)
import functools

import jax
import jax.numpy as jnp
from jax import lax
from jax.experimental import pallas as pl
from jax.experimental.pallas import tpu as pltpu, tpu_sc as plsc

N = 10000
E = 320000
D = 128

NC = 2
NS = 16
NW = NC * NS
G = E // 128
GPW = G // NW
EXTRA = G - GPW * NW
RPT = N // NS
ZC = 125

_mesh = plsc.VectorSubcoreMesh(
    core_axis_name="c", subcore_axis_name="s", num_cores=NC, num_subcores=NS
)


def _worker_id():
    cid = lax.axis_index("c")
    sid = lax.axis_index("s")
    return cid, sid, sid * NC + cid


@functools.partial(
    pl.kernel,
    out_type=[
        jax.ShapeDtypeStruct((NC, N, 16), jnp.float32),
        jax.ShapeDtypeStruct((NC, N, 16), jnp.float32),
    ],
    mesh=_mesh,
    scratch_types=[
        pltpu.VMEM((128,), jnp.int32),
        pltpu.VMEM((128,), jnp.int32),
        pltpu.VMEM((128, 16), jnp.float32),
        pltpu.VMEM((128, 16), jnp.float32),
        pltpu.VMEM_SHARED((N, 16), jnp.float32),
        pltpu.VMEM_SHARED((N, 16), jnp.float32),
    ],
)
def _degree_kernel(send_hbm, recv_hbm, ones_hbm, zeros_hbm,
                   degs_hbm, degr_hbm,
                   idx_s, idx_r, ones_v, z16, acc_s, acc_r):
    cid, sid, wid = _worker_id()
    pltpu.sync_copy(ones_hbm, ones_v)
    pltpu.sync_copy(zeros_hbm, z16)
    r0 = sid * RPT
    for j in range(RPT // ZC):
        pltpu.sync_copy(z16.at[pl.ds(0, ZC)], acc_s.at[pl.ds(r0 + j * ZC, ZC)])
        pltpu.sync_copy(z16.at[pl.ds(0, ZC)], acc_r.at[pl.ds(r0 + j * ZC, ZC)])
    plsc.subcore_barrier()

    def step(g):
        pltpu.sync_copy(send_hbm.at[g], idx_s)
        pltpu.sync_copy(recv_hbm.at[g], idx_r)
        pltpu.sync_copy(ones_v, acc_s.at[idx_s], add=True)
        pltpu.sync_copy(ones_v, acc_r.at[idx_r], add=True)

    def body(t, carry):
        step(wid * GPW + t)
        return carry

    lax.fori_loop(0, GPW, body, 0)

    @pl.when(wid < EXTRA)
    def _():
        step(NW * GPW + wid)

    plsc.subcore_barrier()
    pltpu.sync_copy(acc_s.at[pl.ds(r0, RPT)], degs_hbm.at[cid].at[pl.ds(r0, RPT)])
    pltpu.sync_copy(acc_r.at[pl.ds(r0, RPT)], degr_hbm.at[cid].at[pl.ds(r0, RPT)])


@functools.partial(
    pl.kernel,
    out_type=jax.ShapeDtypeStruct((NC, N, D), jnp.float32),
    mesh=_mesh,
    scratch_types=[
        pltpu.VMEM((128,), jnp.int32),
        pltpu.VMEM((128,), jnp.int32),
        pltpu.VMEM((128, D), jnp.float32),
        pltpu.VMEM_SHARED((N, D), jnp.float32),
        pltpu.SemaphoreType.DMA,
    ],
)
def _agg_kernel(y_hbm, send_hbm, recv_hbm, zeros_hbm, out_hbm,
                idx_s, idx_r, rows, acc, sem):
    cid, sid, wid = _worker_id()
    pltpu.sync_copy(zeros_hbm, rows)
    r0 = sid * RPT
    for j in range(RPT // ZC):
        pltpu.sync_copy(rows.at[pl.ds(0, ZC)], acc.at[pl.ds(r0 + j * ZC, ZC)])
    plsc.subcore_barrier()

    def step(g):
        pltpu.sync_copy(send_hbm.at[g], idx_s)
        pltpu.sync_copy(recv_hbm.at[g], idx_r)
        pltpu.async_copy(y_hbm.at[idx_s], rows, sem).wait()
        pltpu.sync_copy(rows, acc.at[idx_r], add=True)

    def body(t, carry):
        step(wid * GPW + t)
        return carry

    lax.fori_loop(0, GPW, body, 0)

    @pl.when(wid < EXTRA)
    def _():
        step(NW * GPW + wid)

    plsc.subcore_barrier()
    pltpu.sync_copy(acc.at[pl.ds(r0, RPT)], out_hbm.at[cid].at[pl.ds(r0, RPT)])


def _scale_body(ds_ref, dr_ref, ss_ref, sr_ref):
    ds = ds_ref[0, :, :1] + ds_ref[1, :, :1]
    dr = dr_ref[0, :, :1] + dr_ref[1, :, :1]
    ss_ref[...] = lax.rsqrt(jnp.maximum(ds, 1.0))
    sr_ref[...] = lax.rsqrt(jnp.maximum(dr, 1.0))


_scales = pl.pallas_call(
    _scale_body,
    grid=(10,),
    in_specs=[
        pl.BlockSpec((NC, 1000, 16), lambda i: (0, i, 0)),
        pl.BlockSpec((NC, 1000, 16), lambda i: (0, i, 0)),
    ],
    out_specs=[
        pl.BlockSpec((1000, 1), lambda i: (i, 0)),
        pl.BlockSpec((1000, 1), lambda i: (i, 0)),
    ],
    out_shape=[
        jax.ShapeDtypeStruct((N, 1), jnp.float32),
        jax.ShapeDtypeStruct((N, 1), jnp.float32),
    ],
)


def _mm1_body(x_ref, w_ref, b_ref, so_ref, y_ref):
    h = jnp.dot(x_ref[...], w_ref[...], preferred_element_type=jnp.float32)
    h = jnp.maximum(h + b_ref[...], 0.0)
    y_ref[...] = h * so_ref[...]


_mm1 = pl.pallas_call(
    _mm1_body,
    grid=(10,),
    in_specs=[
        pl.BlockSpec((1000, D), lambda i: (i, 0)),
        pl.BlockSpec((D, D), lambda i: (0, 0)),
        pl.BlockSpec((1, D), lambda i: (0, 0)),
        pl.BlockSpec((1000, 1), lambda i: (i, 0)),
    ],
    out_specs=pl.BlockSpec((1000, D), lambda i: (i, 0)),
    out_shape=jax.ShapeDtypeStruct((N, D), jnp.float32),
)


def _mm_body(act, p_ref, si_ref, w_ref, b_ref, so_ref, y_ref):
    x = (p_ref[0] + p_ref[1]) * si_ref[...]
    h = jnp.dot(x, w_ref[...], preferred_element_type=jnp.float32) + b_ref[...]
    if act:
        h = jnp.maximum(h, 0.0)
    y_ref[...] = h * so_ref[...]


def _make_mm(act):
    return pl.pallas_call(
        functools.partial(_mm_body, act),
        grid=(10,),
        in_specs=[
            pl.BlockSpec((NC, 1000, D), lambda i: (0, i, 0)),
            pl.BlockSpec((1000, 1), lambda i: (i, 0)),
            pl.BlockSpec((D, D), lambda i: (0, 0)),
            pl.BlockSpec((1, D), lambda i: (0, 0)),
            pl.BlockSpec((1000, 1), lambda i: (i, 0)),
        ],
        out_specs=pl.BlockSpec((1000, D), lambda i: (i, 0)),
        out_shape=jax.ShapeDtypeStruct((N, D), jnp.float32),
    )


_mm_act = _make_mm(True)
_mm_noact = _make_mm(False)


def _final_body(p_ref, sr_ref, o_ref):
    o_ref[...] = (p_ref[0] + p_ref[1]) * sr_ref[...]


_final = pl.pallas_call(
    _final_body,
    grid=(10,),
    in_specs=[
        pl.BlockSpec((NC, 1000, D), lambda i: (0, i, 0)),
        pl.BlockSpec((1000, 1), lambda i: (i, 0)),
    ],
    out_specs=pl.BlockSpec((1000, D), lambda i: (i, 0)),
    out_shape=jax.ShapeDtypeStruct((N, D), jnp.float32),
)


def kernel(nodes, senders, receivers, W_in, b_in, W_h0, b_h0, W_h1, b_h1,
           W_out, b_out):
    send2d = senders.reshape(G, 128)
    recv2d = receivers.reshape(G, 128)
    ones16 = jnp.ones((128, 16), jnp.float32)
    zeros16 = jnp.zeros((128, 16), jnp.float32)
    zerosD = jnp.zeros((128, D), jnp.float32)

    degs, degr = _degree_kernel(send2d, recv2d, ones16, zeros16)
    s_send, s_recv = _scales(degs, degr)

    y = _mm1(nodes, W_in, b_in.reshape(1, D), s_send)
    p = _agg_kernel(y, send2d, recv2d, zerosD)
    y = _mm_act(p, s_recv, W_h0, b_h0.reshape(1, D), s_send)
    p = _agg_kernel(y, send2d, recv2d, zerosD)
    y = _mm_act(p, s_recv, W_h1, b_h1.reshape(1, D), s_send)
    p = _agg_kernel(y, send2d, recv2d, zerosD)
    y = _mm_noact(p, s_recv, W_out, b_out.reshape(1, D), s_send)
    p = _agg_kernel(y, send2d, recv2d, zerosD)
    return _final(p, s_recv)


# speedup vs baseline: 4.9789x; 4.9789x over previous
import functools

import jax
import jax.numpy as jnp
from jax import lax
from jax.experimental import pallas as pl
from jax.experimental.pallas import tpu as pltpu, tpu_sc as plsc

N = 10000
E = 320000
D = 128

NC = 2
NS = 16
NW = NC * NS
G = E // 128
GPW = G // NW
EXTRA = G - GPW * NW
RPT = 640

_mesh = plsc.VectorSubcoreMesh(
    core_axis_name="c", subcore_axis_name="s", num_cores=NC, num_subcores=NS
)


def _worker_id():
    cid = lax.axis_index("c")
    sid = lax.axis_index("s")
    return cid, sid, sid * NC + cid


def _zero_rows(zbuf, acc, sid):
    base = sid * RPT
    for off, ln in ((0, 128), (128, 128), (256, 128), (384, 16)):
        pltpu.sync_copy(zbuf.at[pl.ds(0, ln)], acc.at[pl.ds(base + off, ln)])

    @pl.when(sid < NS - 1)
    def _():
        for off, ln in ((400, 128), (528, 112)):
            pltpu.sync_copy(zbuf.at[pl.ds(0, ln)], acc.at[pl.ds(base + off, ln)])


def _flush_rows(acc, dst, sid):
    base = sid * RPT
    pltpu.sync_copy(acc.at[pl.ds(base, 400)], dst.at[pl.ds(base, 400)])

    @pl.when(sid < NS - 1)
    def _():
        pltpu.sync_copy(acc.at[pl.ds(base + 400, 240)],
                        dst.at[pl.ds(base + 400, 240)])


@functools.partial(
    pl.kernel,
    out_type=[
        jax.ShapeDtypeStruct((NC, N, D), jnp.float32),
        jax.ShapeDtypeStruct((NC, N, D), jnp.float32),
    ],
    mesh=_mesh,
    scratch_types=[
        pltpu.VMEM((1, 128), jnp.int32),
        pltpu.VMEM((128, D), jnp.float32),
        pltpu.VMEM((128, D), jnp.float32),
        pltpu.VMEM_SHARED((N, D), jnp.float32),
    ],
)
def _degree_kernel(send_hbm, recv_hbm, ones_hbm, zeros_hbm,
                   degs_hbm, degr_hbm,
                   idx, ones_v, zv, acc):
    cid, sid, wid = _worker_id()
    pltpu.sync_copy(ones_hbm, ones_v)
    pltpu.sync_copy(zeros_hbm, zv)

    def one_pass(e_hbm, out_hbm):
        _zero_rows(zv, acc, sid)
        plsc.subcore_barrier()

        def step(g):
            pltpu.sync_copy(e_hbm.at[g], idx)
            pltpu.sync_copy(ones_v, acc.at[idx.at[0]], add=True)

        def body(t, carry):
            step(wid * GPW + t)
            return carry

        lax.fori_loop(0, GPW, body, 0)

        @pl.when(wid < EXTRA)
        def _():
            step(NW * GPW + wid)

        plsc.subcore_barrier()
        _flush_rows(acc, out_hbm.at[cid], sid)
        plsc.subcore_barrier()

    one_pass(send_hbm, degs_hbm)
    one_pass(recv_hbm, degr_hbm)


@functools.partial(
    pl.kernel,
    out_type=jax.ShapeDtypeStruct((NC, N, D), jnp.float32),
    mesh=_mesh,
    scratch_types=[
        pltpu.VMEM((1, 128), jnp.int32),
        pltpu.VMEM((1, 128), jnp.int32),
        pltpu.VMEM((128, D), jnp.float32),
        pltpu.VMEM_SHARED((N, D), jnp.float32),
        pltpu.SemaphoreType.DMA,
    ],
)
def _agg_kernel(y_hbm, send_hbm, recv_hbm, zeros_hbm, out_hbm,
                idx_s, idx_r, rows, acc, sem):
    cid, sid, wid = _worker_id()
    pltpu.sync_copy(zeros_hbm, rows)
    _zero_rows(rows, acc, sid)
    plsc.subcore_barrier()

    def step(g):
        pltpu.sync_copy(send_hbm.at[g], idx_s)
        pltpu.sync_copy(recv_hbm.at[g], idx_r)
        pltpu.async_copy(y_hbm.at[idx_s.at[0]], rows, sem).wait()
        pltpu.sync_copy(rows, acc.at[idx_r.at[0]], add=True)

    def body(t, carry):
        step(wid * GPW + t)
        return carry

    lax.fori_loop(0, GPW, body, 0)

    @pl.when(wid < EXTRA)
    def _():
        step(NW * GPW + wid)

    plsc.subcore_barrier()
    _flush_rows(acc, out_hbm.at[cid], sid)


def _scale_body(ds_ref, dr_ref, ss_ref, sr_ref):
    ds = ds_ref[0, :, :1] + ds_ref[1, :, :1]
    dr = dr_ref[0, :, :1] + dr_ref[1, :, :1]
    ss_ref[...] = lax.rsqrt(jnp.maximum(ds, 1.0))
    sr_ref[...] = lax.rsqrt(jnp.maximum(dr, 1.0))


_scales = pl.pallas_call(
    _scale_body,
    grid=(10,),
    in_specs=[
        pl.BlockSpec((NC, 1000, D), lambda i: (0, i, 0)),
        pl.BlockSpec((NC, 1000, D), lambda i: (0, i, 0)),
    ],
    out_specs=[
        pl.BlockSpec((1000, 1), lambda i: (i, 0)),
        pl.BlockSpec((1000, 1), lambda i: (i, 0)),
    ],
    out_shape=[
        jax.ShapeDtypeStruct((N, 1), jnp.float32),
        jax.ShapeDtypeStruct((N, 1), jnp.float32),
    ],
)


def _mm1_body(x_ref, w_ref, b_ref, so_ref, y_ref):
    h = jnp.dot(x_ref[...], w_ref[...], preferred_element_type=jnp.float32)
    h = jnp.maximum(h + b_ref[...], 0.0)
    y_ref[...] = h * so_ref[...]


_mm1 = pl.pallas_call(
    _mm1_body,
    grid=(10,),
    in_specs=[
        pl.BlockSpec((1000, D), lambda i: (i, 0)),
        pl.BlockSpec((D, D), lambda i: (0, 0)),
        pl.BlockSpec((1, D), lambda i: (0, 0)),
        pl.BlockSpec((1000, 1), lambda i: (i, 0)),
    ],
    out_specs=pl.BlockSpec((1000, D), lambda i: (i, 0)),
    out_shape=jax.ShapeDtypeStruct((N, D), jnp.float32),
)


def _mm_body(act, p_ref, si_ref, w_ref, b_ref, so_ref, y_ref):
    x = (p_ref[0] + p_ref[1]) * si_ref[...]
    h = jnp.dot(x, w_ref[...], preferred_element_type=jnp.float32) + b_ref[...]
    if act:
        h = jnp.maximum(h, 0.0)
    y_ref[...] = h * so_ref[...]


def _make_mm(act):
    return pl.pallas_call(
        functools.partial(_mm_body, act),
        grid=(10,),
        in_specs=[
            pl.BlockSpec((NC, 1000, D), lambda i: (0, i, 0)),
            pl.BlockSpec((1000, 1), lambda i: (i, 0)),
            pl.BlockSpec((D, D), lambda i: (0, 0)),
            pl.BlockSpec((1, D), lambda i: (0, 0)),
            pl.BlockSpec((1000, 1), lambda i: (i, 0)),
        ],
        out_specs=pl.BlockSpec((1000, D), lambda i: (i, 0)),
        out_shape=jax.ShapeDtypeStruct((N, D), jnp.float32),
    )


_mm_act = _make_mm(True)
_mm_noact = _make_mm(False)


def _final_body(p_ref, sr_ref, o_ref):
    o_ref[...] = (p_ref[0] + p_ref[1]) * sr_ref[...]


_final = pl.pallas_call(
    _final_body,
    grid=(10,),
    in_specs=[
        pl.BlockSpec((NC, 1000, D), lambda i: (0, i, 0)),
        pl.BlockSpec((1000, 1), lambda i: (i, 0)),
    ],
    out_specs=pl.BlockSpec((1000, D), lambda i: (i, 0)),
    out_shape=jax.ShapeDtypeStruct((N, D), jnp.float32),
)


def kernel(nodes, senders, receivers, W_in, b_in, W_h0, b_h0, W_h1, b_h1,
           W_out, b_out):
    send3d = senders.reshape(G, 1, 128)
    recv3d = receivers.reshape(G, 1, 128)
    onesD = jnp.ones((128, D), jnp.float32)
    zerosD = jnp.zeros((128, D), jnp.float32)

    degs, degr = _degree_kernel(send3d, recv3d, onesD, zerosD)
    s_send, s_recv = _scales(degs, degr)

    y = _mm1(nodes, W_in, b_in.reshape(1, D), s_send)
    p = _agg_kernel(y, send3d, recv3d, zerosD)
    y = _mm_act(p, s_recv, W_h0, b_h0.reshape(1, D), s_send)
    p = _agg_kernel(y, send3d, recv3d, zerosD)
    y = _mm_act(p, s_recv, W_h1, b_h1.reshape(1, D), s_send)
    p = _agg_kernel(y, send3d, recv3d, zerosD)
    y = _mm_noact(p, s_recv, W_out, b_out.reshape(1, D), s_send)
    p = _agg_kernel(y, send3d, recv3d, zerosD)
    return _final(p, s_recv)


# speedup vs baseline: 7.5976x; 1.5260x over previous
import functools

import jax
import jax.numpy as jnp
from jax import lax
from jax.experimental import pallas as pl
from jax.experimental.pallas import tpu as pltpu, tpu_sc as plsc

N = 10000
E = 320000
D = 128

NC = 2
NS = 16
NW = NC * NS
G = E // 128
GPW = G // NW
EXTRA = G - GPW * NW
RPT = 640

_mesh = plsc.VectorSubcoreMesh(
    core_axis_name="c", subcore_axis_name="s", num_cores=NC, num_subcores=NS
)


def _worker_id():
    cid = lax.axis_index("c")
    sid = lax.axis_index("s")
    return cid, sid, sid * NC + cid


def _zero_rows(zbuf, acc, sid):
    base = sid * RPT
    for off, ln in ((0, 128), (128, 128), (256, 128), (384, 16)):
        pltpu.sync_copy(zbuf.at[pl.ds(0, ln)], acc.at[pl.ds(base + off, ln)])

    @pl.when(sid < NS - 1)
    def _():
        for off, ln in ((400, 128), (528, 112)):
            pltpu.sync_copy(zbuf.at[pl.ds(0, ln)], acc.at[pl.ds(base + off, ln)])


def _flush_rows(acc, dst, sid):
    base = sid * RPT
    pltpu.sync_copy(acc.at[pl.ds(base, 400)], dst.at[pl.ds(base, 400)])

    @pl.when(sid < NS - 1)
    def _():
        pltpu.sync_copy(acc.at[pl.ds(base + 400, 240)],
                        dst.at[pl.ds(base + 400, 240)])


DW = 128

@functools.partial(
    pl.kernel,
    out_type=[
        jax.ShapeDtypeStruct((NC, N, DW), jnp.float32),
        jax.ShapeDtypeStruct((NC, N, DW), jnp.float32),
    ],
    mesh=_mesh,
    scratch_types=[
        pltpu.VMEM((GPW + 1, 1, 128), jnp.int32),
        pltpu.VMEM((128, DW), jnp.float32),
        pltpu.VMEM((128, DW), jnp.float32),
        pltpu.VMEM_SHARED((N, DW), jnp.float32),
        pltpu.SemaphoreType.DMA,
    ],
)
def _degree_kernel(send_hbm, recv_hbm, ones_hbm, zeros_hbm,
                   degs_hbm, degr_hbm,
                   idx, ones_v, zv, acc, sem):
    cid, sid, wid = _worker_id()
    pltpu.sync_copy(ones_hbm, ones_v)
    pltpu.sync_copy(zeros_hbm, zv)
    w0 = wid * GPW

    def one_pass(e_hbm, out_hbm):
        _zero_rows(zv, acc, sid)
        pltpu.sync_copy(e_hbm.at[pl.ds(w0, GPW + 1)], idx)

        @pl.when(wid < EXTRA)
        def _():
            pltpu.sync_copy(e_hbm.at[NW * GPW + wid], idx.at[GPW])

        plsc.subcore_barrier()

        def body(k, carry):
            t0 = 6 * k
            ds = [
                pltpu.async_copy(ones_v, acc.at[idx.at[t0 + j, 0]], sem,
                                 add=True)
                for j in range(6)
            ]
            for d in ds:
                d.wait()
            return carry

        lax.fori_loop(0, GPW // 6, body, 0)

        @pl.when(wid < EXTRA)
        def _():
            pltpu.async_copy(ones_v, acc.at[idx.at[GPW, 0]], sem,
                             add=True).wait()

        plsc.subcore_barrier()
        _flush_rows(acc, out_hbm.at[cid], sid)
        plsc.subcore_barrier()

    one_pass(send_hbm, degs_hbm)
    one_pass(recv_hbm, degr_hbm)


CH = 26

@functools.partial(
    pl.kernel,
    out_type=jax.ShapeDtypeStruct((NC, N, D), jnp.float32),
    mesh=_mesh,
    scratch_types=[
        pltpu.VMEM((CH, 1, 128), jnp.int32),
        pltpu.VMEM((CH, 1, 128), jnp.int32),
        pltpu.VMEM((128, D), jnp.float32),
        pltpu.VMEM((128, D), jnp.float32),
        pltpu.VMEM_SHARED((N, D), jnp.float32),
        pltpu.SemaphoreType.DMA,
        pltpu.SemaphoreType.DMA,
        pltpu.SemaphoreType.DMA,
        pltpu.SemaphoreType.DMA,
    ],
)
def _agg_kernel(y_hbm, send_hbm, recv_hbm, zeros_hbm, out_hbm,
                idx_s, idx_r, rA, rB, acc, gA, gB, sA, sB):
    cid, sid, wid = _worker_id()
    pltpu.sync_copy(zeros_hbm, rA)
    _zero_rows(rA, acc, sid)
    w0 = wid * GPW
    plsc.subcore_barrier()

    def pair(t0):
        dg0 = pltpu.async_copy(y_hbm.at[idx_s.at[t0, 0]], rA, gA)
        dg1 = pltpu.async_copy(y_hbm.at[idx_s.at[t0 + 1, 0]], rB, gB)
        dg0.wait()
        ds0 = pltpu.async_copy(rA, acc.at[idx_r.at[t0, 0]], sA, add=True)
        dg1.wait()
        ds1 = pltpu.async_copy(rB, acc.at[idx_r.at[t0 + 1, 0]], sB, add=True)
        ds0.wait()
        ds1.wait()

    for c in range(GPW // CH):
        pltpu.sync_copy(send_hbm.at[pl.ds(w0 + c * CH, CH)], idx_s)
        pltpu.sync_copy(recv_hbm.at[pl.ds(w0 + c * CH, CH)], idx_r)

        def body(k, carry):
            pair(2 * k)
            return carry

        lax.fori_loop(0, CH // 2, body, 0)

    @pl.when(wid < EXTRA)
    def _():
        pltpu.sync_copy(send_hbm.at[NW * GPW + wid], idx_s.at[0])
        pltpu.sync_copy(recv_hbm.at[NW * GPW + wid], idx_r.at[0])
        pltpu.async_copy(y_hbm.at[idx_s.at[0, 0]], rA, gA).wait()
        pltpu.async_copy(rA, acc.at[idx_r.at[0, 0]], sA, add=True).wait()

    plsc.subcore_barrier()
    _flush_rows(acc, out_hbm.at[cid], sid)


def _scale_body(ds_ref, dr_ref, ss_ref, sr_ref):
    ds = ds_ref[0, :, :1] + ds_ref[1, :, :1]
    dr = dr_ref[0, :, :1] + dr_ref[1, :, :1]
    ss_ref[...] = lax.rsqrt(jnp.maximum(ds, 1.0))
    sr_ref[...] = lax.rsqrt(jnp.maximum(dr, 1.0))


_scales = pl.pallas_call(
    _scale_body,
    grid=(10,),
    in_specs=[
        pl.BlockSpec((NC, 1000, DW), lambda i: (0, i, 0)),
        pl.BlockSpec((NC, 1000, DW), lambda i: (0, i, 0)),
    ],
    out_specs=[
        pl.BlockSpec((1000, 1), lambda i: (i, 0)),
        pl.BlockSpec((1000, 1), lambda i: (i, 0)),
    ],
    out_shape=[
        jax.ShapeDtypeStruct((N, 1), jnp.float32),
        jax.ShapeDtypeStruct((N, 1), jnp.float32),
    ],
)


def _mm1_body(x_ref, w_ref, b_ref, so_ref, y_ref):
    h = jnp.dot(x_ref[...], w_ref[...], preferred_element_type=jnp.float32)
    h = jnp.maximum(h + b_ref[...], 0.0)
    y_ref[...] = h * so_ref[...]


_mm1 = pl.pallas_call(
    _mm1_body,
    grid=(10,),
    in_specs=[
        pl.BlockSpec((1000, D), lambda i: (i, 0)),
        pl.BlockSpec((D, D), lambda i: (0, 0)),
        pl.BlockSpec((1, D), lambda i: (0, 0)),
        pl.BlockSpec((1000, 1), lambda i: (i, 0)),
    ],
    out_specs=pl.BlockSpec((1000, D), lambda i: (i, 0)),
    out_shape=jax.ShapeDtypeStruct((N, D), jnp.float32),
)


def _mm_body(act, p_ref, si_ref, w_ref, b_ref, so_ref, y_ref):
    x = (p_ref[0] + p_ref[1]) * si_ref[...]
    h = jnp.dot(x, w_ref[...], preferred_element_type=jnp.float32) + b_ref[...]
    if act:
        h = jnp.maximum(h, 0.0)
    y_ref[...] = h * so_ref[...]


def _make_mm(act):
    return pl.pallas_call(
        functools.partial(_mm_body, act),
        grid=(10,),
        in_specs=[
            pl.BlockSpec((NC, 1000, D), lambda i: (0, i, 0)),
            pl.BlockSpec((1000, 1), lambda i: (i, 0)),
            pl.BlockSpec((D, D), lambda i: (0, 0)),
            pl.BlockSpec((1, D), lambda i: (0, 0)),
            pl.BlockSpec((1000, 1), lambda i: (i, 0)),
        ],
        out_specs=pl.BlockSpec((1000, D), lambda i: (i, 0)),
        out_shape=jax.ShapeDtypeStruct((N, D), jnp.float32),
    )


_mm_act = _make_mm(True)
_mm_noact = _make_mm(False)


def _final_body(p_ref, sr_ref, o_ref):
    o_ref[...] = (p_ref[0] + p_ref[1]) * sr_ref[...]


_final = pl.pallas_call(
    _final_body,
    grid=(10,),
    in_specs=[
        pl.BlockSpec((NC, 1000, D), lambda i: (0, i, 0)),
        pl.BlockSpec((1000, 1), lambda i: (i, 0)),
    ],
    out_specs=pl.BlockSpec((1000, D), lambda i: (i, 0)),
    out_shape=jax.ShapeDtypeStruct((N, D), jnp.float32),
)


def kernel(nodes, senders, receivers, W_in, b_in, W_h0, b_h0, W_h1, b_h1,
           W_out, b_out):
    send3d = senders.reshape(G, 1, 128)
    recv3d = receivers.reshape(G, 1, 128)
    onesW = jnp.ones((128, DW), jnp.float32)
    zerosW = jnp.zeros((128, DW), jnp.float32)
    zerosD = jnp.zeros((128, D), jnp.float32)

    degs, degr = _degree_kernel(send3d, recv3d, onesW, zerosW)
    s_send, s_recv = _scales(degs, degr)

    y = _mm1(nodes, W_in, b_in.reshape(1, D), s_send)
    p = _agg_kernel(y, send3d, recv3d, zerosD)
    y = _mm_act(p, s_recv, W_h0, b_h0.reshape(1, D), s_send)
    p = _agg_kernel(y, send3d, recv3d, zerosD)
    y = _mm_act(p, s_recv, W_h1, b_h1.reshape(1, D), s_send)
    p = _agg_kernel(y, send3d, recv3d, zerosD)
    y = _mm_noact(p, s_recv, W_out, b_out.reshape(1, D), s_send)
    p = _agg_kernel(y, send3d, recv3d, zerosD)
    return _final(p, s_recv)


# speedup vs baseline: 7.6956x; 1.0129x over previous
import functools

import jax
import jax.numpy as jnp
from jax import lax
from jax.experimental import pallas as pl
from jax.experimental.pallas import tpu as pltpu, tpu_sc as plsc

N = 10000
E = 320000
D = 128

NC = 2
NS = 16
NW = NC * NS
G = E // 128
GPW = G // NW
EXTRA = G - GPW * NW
RPT = 640

_mesh = plsc.VectorSubcoreMesh(
    core_axis_name="c", subcore_axis_name="s", num_cores=NC, num_subcores=NS
)


def _worker_id():
    cid = lax.axis_index("c")
    sid = lax.axis_index("s")
    return cid, sid, sid * NC + cid


def _zero_rows(zbuf, acc, sid):
    base = sid * RPT
    for off, ln in ((0, 128), (128, 128), (256, 128), (384, 16)):
        pltpu.sync_copy(zbuf.at[pl.ds(0, ln)], acc.at[pl.ds(base + off, ln)])

    @pl.when(sid < NS - 1)
    def _():
        for off, ln in ((400, 128), (528, 112)):
            pltpu.sync_copy(zbuf.at[pl.ds(0, ln)], acc.at[pl.ds(base + off, ln)])


def _flush_rows(acc, dst, sid):
    base = sid * RPT
    pltpu.sync_copy(acc.at[pl.ds(base, 400)], dst.at[pl.ds(base, 400)])

    @pl.when(sid < NS - 1)
    def _():
        pltpu.sync_copy(acc.at[pl.ds(base + 400, 240)],
                        dst.at[pl.ds(base + 400, 240)])


DW = 128

@functools.partial(
    pl.kernel,
    out_type=[
        jax.ShapeDtypeStruct((NC, N, DW), jnp.float32),
        jax.ShapeDtypeStruct((NC, N, DW), jnp.float32),
    ],
    mesh=_mesh,
    scratch_types=[
        pltpu.VMEM((GPW + 1, 1, 128), jnp.int32),
        pltpu.VMEM((128, DW), jnp.float32),
        pltpu.VMEM((128, DW), jnp.float32),
        pltpu.VMEM_SHARED((N, DW), jnp.float32),
        pltpu.SemaphoreType.DMA,
    ],
)
def _degree_kernel(send_hbm, recv_hbm, ones_hbm, zeros_hbm,
                   degs_hbm, degr_hbm,
                   idx, ones_v, zv, acc, sem):
    cid, sid, wid = _worker_id()
    pltpu.sync_copy(ones_hbm, ones_v)
    pltpu.sync_copy(zeros_hbm, zv)
    w0 = wid * GPW

    def one_pass(e_hbm, out_hbm):
        _zero_rows(zv, acc, sid)
        pltpu.sync_copy(e_hbm.at[pl.ds(w0, GPW + 1)], idx)

        @pl.when(wid < EXTRA)
        def _():
            pltpu.sync_copy(e_hbm.at[NW * GPW + wid], idx.at[GPW])

        plsc.subcore_barrier()

        def body(k, carry):
            t0 = 6 * k
            ds = [
                pltpu.async_copy(ones_v, acc.at[idx.at[t0 + j, 0]], sem,
                                 add=True)
                for j in range(6)
            ]
            for d in ds:
                d.wait()
            return carry

        lax.fori_loop(0, GPW // 6, body, 0)

        @pl.when(wid < EXTRA)
        def _():
            pltpu.async_copy(ones_v, acc.at[idx.at[GPW, 0]], sem,
                             add=True).wait()

        plsc.subcore_barrier()
        _flush_rows(acc, out_hbm.at[cid], sid)
        plsc.subcore_barrier()

    one_pass(send_hbm, degs_hbm)
    one_pass(recv_hbm, degr_hbm)


CH = 26

@functools.partial(
    pl.kernel,
    out_type=jax.ShapeDtypeStruct((NC, N, D), jnp.float32),
    mesh=_mesh,
    scratch_types=[
        pltpu.VMEM((CH, 1, 128), jnp.int32),
        pltpu.VMEM((CH, 1, 128), jnp.int32),
        pltpu.VMEM((128, D), jnp.float32),
        pltpu.VMEM((128, D), jnp.float32),
        pltpu.VMEM_SHARED((N, D), jnp.float32),
        pltpu.SemaphoreType.DMA,
        pltpu.SemaphoreType.DMA,
        pltpu.SemaphoreType.DMA,
        pltpu.SemaphoreType.DMA,
    ],
)
def _agg_kernel(y_hbm, send_hbm, recv_hbm, zeros_hbm, out_hbm,
                idx_s, idx_r, rA, rB, acc, gA, gB, sA, sB):
    cid, sid, wid = _worker_id()
    pltpu.sync_copy(zeros_hbm, rA)
    _zero_rows(rA, acc, sid)
    w0 = wid * GPW
    plsc.subcore_barrier()

    def drainA():
        pltpu.make_async_copy(rA, acc.at[idx_r.at[0, 0]], sA).wait()

    def drainB():
        pltpu.make_async_copy(rB, acc.at[idx_r.at[0, 0]], sB).wait()

    def pair(t0, pre_drain):
        if pre_drain:
            drainA()
        dg0 = pltpu.async_copy(y_hbm.at[idx_s.at[t0, 0]], rA, gA)
        if pre_drain:
            drainB()
        dg1 = pltpu.async_copy(y_hbm.at[idx_s.at[t0 + 1, 0]], rB, gB)
        dg0.wait()
        pltpu.async_copy(rA, acc.at[idx_r.at[t0, 0]], sA, add=True)
        dg1.wait()
        pltpu.async_copy(rB, acc.at[idx_r.at[t0 + 1, 0]], sB, add=True)

    for c in range(GPW // CH):
        pltpu.sync_copy(send_hbm.at[pl.ds(w0 + c * CH, CH)], idx_s)
        pltpu.sync_copy(recv_hbm.at[pl.ds(w0 + c * CH, CH)], idx_r)
        pair(0, False)

        def body(k, carry):
            pair(2 * k, True)
            return carry

        lax.fori_loop(1, CH // 2, body, 0)
        drainA()
        drainB()

    @pl.when(wid < EXTRA)
    def _():
        pltpu.sync_copy(send_hbm.at[NW * GPW + wid], idx_s.at[0])
        pltpu.sync_copy(recv_hbm.at[NW * GPW + wid], idx_r.at[0])
        pltpu.async_copy(y_hbm.at[idx_s.at[0, 0]], rA, gA).wait()
        pltpu.async_copy(rA, acc.at[idx_r.at[0, 0]], sA, add=True).wait()

    plsc.subcore_barrier()
    _flush_rows(acc, out_hbm.at[cid], sid)


def _scale_body(ds_ref, dr_ref, ss_ref, sr_ref):
    ds = ds_ref[0, :, :1] + ds_ref[1, :, :1]
    dr = dr_ref[0, :, :1] + dr_ref[1, :, :1]
    ss_ref[...] = lax.rsqrt(jnp.maximum(ds, 1.0))
    sr_ref[...] = lax.rsqrt(jnp.maximum(dr, 1.0))


_scales = pl.pallas_call(
    _scale_body,
    grid=(10,),
    in_specs=[
        pl.BlockSpec((NC, 1000, DW), lambda i: (0, i, 0)),
        pl.BlockSpec((NC, 1000, DW), lambda i: (0, i, 0)),
    ],
    out_specs=[
        pl.BlockSpec((1000, 1), lambda i: (i, 0)),
        pl.BlockSpec((1000, 1), lambda i: (i, 0)),
    ],
    out_shape=[
        jax.ShapeDtypeStruct((N, 1), jnp.float32),
        jax.ShapeDtypeStruct((N, 1), jnp.float32),
    ],
)


def _mm1_body(x_ref, w_ref, b_ref, so_ref, y_ref):
    h = jnp.dot(x_ref[...], w_ref[...], preferred_element_type=jnp.float32)
    h = jnp.maximum(h + b_ref[...], 0.0)
    y_ref[...] = h * so_ref[...]


_mm1 = pl.pallas_call(
    _mm1_body,
    grid=(10,),
    in_specs=[
        pl.BlockSpec((1000, D), lambda i: (i, 0)),
        pl.BlockSpec((D, D), lambda i: (0, 0)),
        pl.BlockSpec((1, D), lambda i: (0, 0)),
        pl.BlockSpec((1000, 1), lambda i: (i, 0)),
    ],
    out_specs=pl.BlockSpec((1000, D), lambda i: (i, 0)),
    out_shape=jax.ShapeDtypeStruct((N, D), jnp.float32),
)


def _mm_body(act, p_ref, si_ref, w_ref, b_ref, so_ref, y_ref):
    x = (p_ref[0] + p_ref[1]) * si_ref[...]
    h = jnp.dot(x, w_ref[...], preferred_element_type=jnp.float32) + b_ref[...]
    if act:
        h = jnp.maximum(h, 0.0)
    y_ref[...] = h * so_ref[...]


def _make_mm(act):
    return pl.pallas_call(
        functools.partial(_mm_body, act),
        grid=(10,),
        in_specs=[
            pl.BlockSpec((NC, 1000, D), lambda i: (0, i, 0)),
            pl.BlockSpec((1000, 1), lambda i: (i, 0)),
            pl.BlockSpec((D, D), lambda i: (0, 0)),
            pl.BlockSpec((1, D), lambda i: (0, 0)),
            pl.BlockSpec((1000, 1), lambda i: (i, 0)),
        ],
        out_specs=pl.BlockSpec((1000, D), lambda i: (i, 0)),
        out_shape=jax.ShapeDtypeStruct((N, D), jnp.float32),
    )


_mm_act = _make_mm(True)
_mm_noact = _make_mm(False)


def _final_body(p_ref, sr_ref, o_ref):
    o_ref[...] = (p_ref[0] + p_ref[1]) * sr_ref[...]


_final = pl.pallas_call(
    _final_body,
    grid=(10,),
    in_specs=[
        pl.BlockSpec((NC, 1000, D), lambda i: (0, i, 0)),
        pl.BlockSpec((1000, 1), lambda i: (i, 0)),
    ],
    out_specs=pl.BlockSpec((1000, D), lambda i: (i, 0)),
    out_shape=jax.ShapeDtypeStruct((N, D), jnp.float32),
)


def kernel(nodes, senders, receivers, W_in, b_in, W_h0, b_h0, W_h1, b_h1,
           W_out, b_out):
    send3d = senders.reshape(G, 1, 128)
    recv3d = receivers.reshape(G, 1, 128)
    onesW = jnp.ones((128, DW), jnp.float32)
    zerosW = jnp.zeros((128, DW), jnp.float32)
    zerosD = jnp.zeros((128, D), jnp.float32)

    degs, degr = _degree_kernel(send3d, recv3d, onesW, zerosW)
    s_send, s_recv = _scales(degs, degr)

    y = _mm1(nodes, W_in, b_in.reshape(1, D), s_send)
    p = _agg_kernel(y, send3d, recv3d, zerosD)
    y = _mm_act(p, s_recv, W_h0, b_h0.reshape(1, D), s_send)
    p = _agg_kernel(y, send3d, recv3d, zerosD)
    y = _mm_act(p, s_recv, W_h1, b_h1.reshape(1, D), s_send)
    p = _agg_kernel(y, send3d, recv3d, zerosD)
    y = _mm_noact(p, s_recv, W_out, b_out.reshape(1, D), s_send)
    p = _agg_kernel(y, send3d, recv3d, zerosD)
    return _final(p, s_recv)


# speedup vs baseline: 7.7729x; 1.0100x over previous
import functools

import jax
import jax.numpy as jnp
from jax import lax
from jax.experimental import pallas as pl
from jax.experimental.pallas import tpu as pltpu, tpu_sc as plsc

N = 10000
E = 320000
D = 128

NC = 2
NS = 16
NW = NC * NS
G = E // 128
GPW = G // NW
EXTRA = G - GPW * NW
RPT = 640

_mesh = plsc.VectorSubcoreMesh(
    core_axis_name="c", subcore_axis_name="s", num_cores=NC, num_subcores=NS
)


def _worker_id():
    cid = lax.axis_index("c")
    sid = lax.axis_index("s")
    return cid, sid, sid * NC + cid


def _zero_rows(zbuf, acc, sid):
    base = sid * RPT
    for off, ln in ((0, 128), (128, 128), (256, 128), (384, 16)):
        pltpu.sync_copy(zbuf.at[pl.ds(0, ln)], acc.at[pl.ds(base + off, ln)])

    @pl.when(sid < NS - 1)
    def _():
        for off, ln in ((400, 128), (528, 112)):
            pltpu.sync_copy(zbuf.at[pl.ds(0, ln)], acc.at[pl.ds(base + off, ln)])


def _flush_rows(acc, dst, sid):
    base = sid * RPT
    pltpu.sync_copy(acc.at[pl.ds(base, 400)], dst.at[pl.ds(base, 400)])

    @pl.when(sid < NS - 1)
    def _():
        pltpu.sync_copy(acc.at[pl.ds(base + 400, 240)],
                        dst.at[pl.ds(base + 400, 240)])


DW = 128

@functools.partial(
    pl.kernel,
    out_type=[
        jax.ShapeDtypeStruct((NC, N, DW), jnp.float32),
        jax.ShapeDtypeStruct((NC, N, DW), jnp.float32),
    ],
    mesh=_mesh,
    scratch_types=[
        pltpu.VMEM((GPW + 1, 1, 128), jnp.int32),
        pltpu.VMEM((128, DW), jnp.float32),
        pltpu.VMEM((128, DW), jnp.float32),
        pltpu.VMEM_SHARED((N, DW), jnp.float32),
        pltpu.SemaphoreType.DMA,
    ],
)
def _degree_kernel(send_hbm, recv_hbm, ones_hbm, zeros_hbm,
                   degs_hbm, degr_hbm,
                   idx, ones_v, zv, acc, sem):
    cid, sid, wid = _worker_id()
    pltpu.sync_copy(ones_hbm, ones_v)
    pltpu.sync_copy(zeros_hbm, zv)
    w0 = wid * GPW

    def one_pass(e_hbm, out_hbm):
        _zero_rows(zv, acc, sid)
        pltpu.sync_copy(e_hbm.at[pl.ds(w0, GPW + 1)], idx)

        @pl.when(wid < EXTRA)
        def _():
            pltpu.sync_copy(e_hbm.at[NW * GPW + wid], idx.at[GPW])

        plsc.subcore_barrier()

        def body(k, carry):
            t0 = 6 * k
            ds = [
                pltpu.async_copy(ones_v, acc.at[idx.at[t0 + j, 0]], sem,
                                 add=True)
                for j in range(6)
            ]
            for d in ds:
                d.wait()
            return carry

        lax.fori_loop(0, GPW // 6, body, 0)

        @pl.when(wid < EXTRA)
        def _():
            pltpu.async_copy(ones_v, acc.at[idx.at[GPW, 0]], sem,
                             add=True).wait()

        plsc.subcore_barrier()
        _flush_rows(acc, out_hbm.at[cid], sid)
        plsc.subcore_barrier()

    one_pass(send_hbm, degs_hbm)
    one_pass(recv_hbm, degr_hbm)


CH = 26

@functools.partial(
    pl.kernel,
    out_type=jax.ShapeDtypeStruct((NC, N, D), jnp.float32),
    mesh=_mesh,
    scratch_types=[
        pltpu.VMEM((CH, 1, 128), jnp.int32),
        pltpu.VMEM((CH, 1, 128), jnp.int32),
        pltpu.VMEM((128, D), jnp.float32),
        pltpu.VMEM((128, D), jnp.float32),
        pltpu.VMEM_SHARED((N, D), jnp.float32),
        pltpu.SemaphoreType.DMA,
        pltpu.SemaphoreType.DMA,
        pltpu.SemaphoreType.DMA,
        pltpu.SemaphoreType.DMA,
    ],
)
def _agg_kernel(y_hbm, send_hbm, recv_hbm, zeros_hbm, out_hbm,
                idx_s, idx_r, rA, rB, acc, gA, gB, sA, sB):
    cid, sid, wid = _worker_id()
    pltpu.sync_copy(zeros_hbm, rA)
    _zero_rows(rA, acc, sid)
    w0 = wid * GPW
    plsc.subcore_barrier()

    def drainA():
        pltpu.make_async_copy(rA, acc.at[idx_r.at[0, 0]], sA).wait()

    def drainB():
        pltpu.make_async_copy(rB, acc.at[idx_r.at[0, 0]], sB).wait()

    def pair(t0, pre_drain):
        if pre_drain:
            drainA()
        dg0 = pltpu.async_copy(y_hbm.at[idx_s.at[t0, 0]], rA, gA)
        if pre_drain:
            drainB()
        dg1 = pltpu.async_copy(y_hbm.at[idx_s.at[t0 + 1, 0]], rB, gB)
        dg0.wait()
        pltpu.async_copy(rA, acc.at[idx_r.at[t0, 0]], sA, add=True)
        dg1.wait()
        pltpu.async_copy(rB, acc.at[idx_r.at[t0 + 1, 0]], sB, add=True)

    for c in range(GPW // CH):
        pltpu.sync_copy(send_hbm.at[pl.ds(w0 + c * CH, CH)], idx_s)
        pltpu.sync_copy(recv_hbm.at[pl.ds(w0 + c * CH, CH)], idx_r)
        pair(0, False)

        def body(k, carry):
            pair(2 * k, True)
            return carry

        lax.fori_loop(1, CH // 2, body, 0)
        drainA()
        drainB()

    @pl.when(wid < EXTRA)
    def _():
        pltpu.sync_copy(send_hbm.at[NW * GPW + wid], idx_s.at[0])
        pltpu.sync_copy(recv_hbm.at[NW * GPW + wid], idx_r.at[0])
        pltpu.async_copy(y_hbm.at[idx_s.at[0, 0]], rA, gA).wait()
        pltpu.async_copy(rA, acc.at[idx_r.at[0, 0]], sA, add=True).wait()

    plsc.subcore_barrier()
    _flush_rows(acc, out_hbm.at[cid], sid)


def _mm1_body(x_ref, w_ref, b_ref, ds_ref, dr_ref, y_ref, ss_ref, sr_ref):
    ss = lax.rsqrt(jnp.maximum(ds_ref[0, :, :1] + ds_ref[1, :, :1], 1.0))
    sr = lax.rsqrt(jnp.maximum(dr_ref[0, :, :1] + dr_ref[1, :, :1], 1.0))
    ss_ref[...] = ss
    sr_ref[...] = sr
    h = jnp.dot(x_ref[...], w_ref[...], preferred_element_type=jnp.float32)
    h = jnp.maximum(h + b_ref[...], 0.0)
    y_ref[...] = h * ss


_mm1 = pl.pallas_call(
    _mm1_body,
    grid=(10,),
    in_specs=[
        pl.BlockSpec((1000, D), lambda i: (i, 0)),
        pl.BlockSpec((D, D), lambda i: (0, 0)),
        pl.BlockSpec((1, D), lambda i: (0, 0)),
        pl.BlockSpec((NC, 1000, DW), lambda i: (0, i, 0)),
        pl.BlockSpec((NC, 1000, DW), lambda i: (0, i, 0)),
    ],
    out_specs=[
        pl.BlockSpec((1000, D), lambda i: (i, 0)),
        pl.BlockSpec((1000, 1), lambda i: (i, 0)),
        pl.BlockSpec((1000, 1), lambda i: (i, 0)),
    ],
    out_shape=[
        jax.ShapeDtypeStruct((N, D), jnp.float32),
        jax.ShapeDtypeStruct((N, 1), jnp.float32),
        jax.ShapeDtypeStruct((N, 1), jnp.float32),
    ],
)


def _mm_body(act, p_ref, si_ref, w_ref, b_ref, so_ref, y_ref):
    x = (p_ref[0] + p_ref[1]) * si_ref[...]
    h = jnp.dot(x, w_ref[...], preferred_element_type=jnp.float32) + b_ref[...]
    if act:
        h = jnp.maximum(h, 0.0)
    y_ref[...] = h * so_ref[...]


def _make_mm(act):
    return pl.pallas_call(
        functools.partial(_mm_body, act),
        grid=(10,),
        in_specs=[
            pl.BlockSpec((NC, 1000, D), lambda i: (0, i, 0)),
            pl.BlockSpec((1000, 1), lambda i: (i, 0)),
            pl.BlockSpec((D, D), lambda i: (0, 0)),
            pl.BlockSpec((1, D), lambda i: (0, 0)),
            pl.BlockSpec((1000, 1), lambda i: (i, 0)),
        ],
        out_specs=pl.BlockSpec((1000, D), lambda i: (i, 0)),
        out_shape=jax.ShapeDtypeStruct((N, D), jnp.float32),
    )


_mm_act = _make_mm(True)
_mm_noact = _make_mm(False)


def _final_body(p_ref, sr_ref, o_ref):
    o_ref[...] = (p_ref[0] + p_ref[1]) * sr_ref[...]


_final = pl.pallas_call(
    _final_body,
    grid=(10,),
    in_specs=[
        pl.BlockSpec((NC, 1000, D), lambda i: (0, i, 0)),
        pl.BlockSpec((1000, 1), lambda i: (i, 0)),
    ],
    out_specs=pl.BlockSpec((1000, D), lambda i: (i, 0)),
    out_shape=jax.ShapeDtypeStruct((N, D), jnp.float32),
)


def kernel(nodes, senders, receivers, W_in, b_in, W_h0, b_h0, W_h1, b_h1,
           W_out, b_out):
    send3d = senders.reshape(G, 1, 128)
    recv3d = receivers.reshape(G, 1, 128)
    onesW = jnp.ones((128, DW), jnp.float32)
    zerosW = jnp.zeros((128, DW), jnp.float32)
    zerosD = jnp.zeros((128, D), jnp.float32)

    degs, degr = _degree_kernel(send3d, recv3d, onesW, zerosW)

    y, s_send, s_recv = _mm1(nodes, W_in, b_in.reshape(1, D), degs, degr)
    p = _agg_kernel(y, send3d, recv3d, zerosD)
    y = _mm_act(p, s_recv, W_h0, b_h0.reshape(1, D), s_send)
    p = _agg_kernel(y, send3d, recv3d, zerosD)
    y = _mm_act(p, s_recv, W_h1, b_h1.reshape(1, D), s_send)
    p = _agg_kernel(y, send3d, recv3d, zerosD)
    y = _mm_noact(p, s_recv, W_out, b_out.reshape(1, D), s_send)
    p = _agg_kernel(y, send3d, recv3d, zerosD)
    return _final(p, s_recv)
